# baseline (speedup 1.0000x reference)
import functools

import jax
import jax.numpy as jnp
import numpy as np
from jax import lax
from jax.experimental import pallas as pl
from jax.experimental.pallas import tpu as pltpu
from jax.experimental.pallas import tpu_sc as plsc

_ALPHA = 4


@functools.lru_cache(maxsize=None)
def _slow_indices(T: int) -> tuple:
    n = T // _ALPHA
    i = np.arange(n - 1, dtype=np.float32) / np.float32(n - 1)
    lo, hi = np.float32(0.0), np.float32(T - 1)
    vals = np.concatenate([lo * (np.float32(1.0) - i) + hi * i, [hi]])
    return tuple(int(v) for v in vals.astype(np.int32))


def _fast_copy(frames):
    C, T, H, W = frames.shape
    CPB = 8
    FT = T // CPB
    NCH = C * CPB
    NBUF = 6
    LOOKAHEAD = 2

    def body(in_hbm, out_hbm, *scratch):
        bufs = scratch[:NBUF]
        rsem, wsem = scratch[NBUF], scratch[NBUF + 1]

        def sl(i):
            return (i // CPB, pl.ds((i % CPB) * FT, FT))

        reads, writes = {}, {}
        for i in range(NCH + LOOKAHEAD):
            if i < NCH:
                b = i % NBUF
                if i >= NBUF:
                    writes[i - NBUF].wait()
                c, ds = sl(i)
                reads[i] = pltpu.make_async_copy(
                    in_hbm.at[c, ds], bufs[b], rsem.at[b]
                )
                reads[i].start()
            d = i - LOOKAHEAD
            if 0 <= d < NCH:
                b = d % NBUF
                c, ds = sl(d)
                reads[d].wait()
                writes[d] = pltpu.make_async_copy(
                    bufs[b], out_hbm.at[c, ds], wsem.at[b]
                )
                writes[d].start()
        for d in range(NCH - NBUF, NCH):
            writes[d].wait()

    return pl.pallas_call(
        body,
        in_specs=[pl.BlockSpec(memory_space=pl.ANY)],
        out_specs=pl.BlockSpec(memory_space=pl.ANY),
        out_shape=jax.ShapeDtypeStruct((C, T, H, W), frames.dtype),
        scratch_shapes=(
            [pltpu.VMEM((FT, H, W), frames.dtype) for _ in range(NBUF)]
            + [pltpu.SemaphoreType.DMA((NBUF,)), pltpu.SemaphoreType.DMA((NBUF,))]
        ),
    )(frames)


def _slow_gather_sc(frames, sel):
    C, T, H, W = frames.shape
    S = len(sel)
    HH = H // 2

    info = plsc.get_sparse_core_info()
    NW = info.num_cores * info.num_subcores
    n_chunks = C * S * 2
    per_w = n_chunks // NW

    mesh = plsc.VectorSubcoreMesh(core_axis_name="c", subcore_axis_name="s")

    def chunk_coords(chunk):
        r = chunk // 2
        half = chunk % 2
        ch = r // S
        k = r % S
        src_t = functools.reduce(
            lambda acc, i: jnp.where(k == i, sel[i], acc),
            range(S),
            jnp.int32(0),
        )
        return ch, k, src_t, half * HH

    @functools.partial(
        pl.kernel,
        out_type=jax.ShapeDtypeStruct((C, S, H, W), frames.dtype),
        mesh=mesh,
        scratch_types=[
            pltpu.VMEM((HH, W), frames.dtype),
            pltpu.VMEM((HH, W), frames.dtype),
            pltpu.VMEM((HH, W), frames.dtype),
            pltpu.SemaphoreType.DMA,
            pltpu.SemaphoreType.DMA,
            pltpu.SemaphoreType.DMA,
        ],
    )
    def gather(frames_hbm, slow_hbm, buf0, buf1, buf2, sem0, sem1, sem2):
        wid = lax.axis_index("s") * info.num_cores + lax.axis_index("c")
        bufs = (buf0, buf1, buf2)
        sems = (sem0, sem1, sem2)
        coords = [chunk_coords(wid * per_w + j) for j in range(per_w)]
        reads = [
            pltpu.async_copy(
                frames_hbm.at[c_, t_, pl.ds(h0, HH)], bufs[j], sems[j]
            )
            for j, (c_, _, t_, h0) in enumerate(coords)
        ]
        writes = []
        for j, (c_, k_, _, h0) in enumerate(coords):
            reads[j].wait()
            writes.append(
                pltpu.async_copy(
                    bufs[j], slow_hbm.at[c_, k_, pl.ds(h0, HH)], sems[j]
                )
            )
        for wr in writes:
            wr.wait()

    return gather(frames)


def kernel(frames):
    T = frames.shape[1]
    sel = _slow_indices(T)
    slow = _slow_gather_sc(frames, sel)
    fast = _fast_copy(frames)
    return (slow, fast)


# speedup vs baseline: 1.0209x; 1.0209x over previous
import functools

import jax
import jax.numpy as jnp
import numpy as np
from jax import lax
from jax.experimental import pallas as pl
from jax.experimental.pallas import tpu as pltpu
from jax.experimental.pallas import tpu_sc as plsc

_ALPHA = 4


@functools.lru_cache(maxsize=None)
def _slow_indices(T: int) -> tuple:
    n = T // _ALPHA
    i = np.arange(n - 1, dtype=np.float32) / np.float32(n - 1)
    lo, hi = np.float32(0.0), np.float32(T - 1)
    vals = np.concatenate([lo * (np.float32(1.0) - i) + hi * i, [hi]])
    return tuple(int(v) for v in vals.astype(np.int32))


def _copy_body(in_ref, out_ref):
    out_ref[...] = in_ref[...]


def _fast_copy(frames):
    C, T, H, W = frames.shape
    BT = 32
    return pl.pallas_call(
        _copy_body,
        grid=(C, T // BT),
        in_specs=[pl.BlockSpec((1, BT, H, W), lambda c, i: (c, i, 0, 0))],
        out_specs=pl.BlockSpec((1, BT, H, W), lambda c, i: (c, i, 0, 0)),
        out_shape=jax.ShapeDtypeStruct((C, T, H, W), frames.dtype),
        compiler_params=pltpu.CompilerParams(
            dimension_semantics=("arbitrary", "arbitrary")
        ),
    )(frames)


def _slow_gather_sc(frames, sel):
    C, T, H, W = frames.shape
    S = len(sel)
    HH = H // 2

    info = plsc.get_sparse_core_info()
    NW = info.num_cores * info.num_subcores
    n_chunks = C * S * 2
    per_w = n_chunks // NW

    mesh = plsc.VectorSubcoreMesh(core_axis_name="c", subcore_axis_name="s")

    def chunk_coords(chunk):
        r = chunk // 2
        half = chunk % 2
        ch = r // S
        k = r % S
        src_t = functools.reduce(
            lambda acc, i: jnp.where(k == i, sel[i], acc),
            range(S),
            jnp.int32(0),
        )
        return ch, k, src_t, half * HH

    @functools.partial(
        pl.kernel,
        out_type=jax.ShapeDtypeStruct((C, S, H, W), frames.dtype),
        mesh=mesh,
        scratch_types=[
            pltpu.VMEM((HH, W), frames.dtype),
            pltpu.VMEM((HH, W), frames.dtype),
            pltpu.VMEM((HH, W), frames.dtype),
            pltpu.SemaphoreType.DMA,
            pltpu.SemaphoreType.DMA,
            pltpu.SemaphoreType.DMA,
        ],
    )
    def gather(frames_hbm, slow_hbm, buf0, buf1, buf2, sem0, sem1, sem2):
        wid = lax.axis_index("s") * info.num_cores + lax.axis_index("c")
        bufs = (buf0, buf1, buf2)
        sems = (sem0, sem1, sem2)
        coords = [chunk_coords(wid * per_w + j) for j in range(per_w)]
        reads = [
            pltpu.async_copy(
                frames_hbm.at[c_, t_, pl.ds(h0, HH)], bufs[j], sems[j]
            )
            for j, (c_, _, t_, h0) in enumerate(coords)
        ]
        writes = []
        for j, (c_, k_, _, h0) in enumerate(coords):
            reads[j].wait()
            writes.append(
                pltpu.async_copy(
                    bufs[j], slow_hbm.at[c_, k_, pl.ds(h0, HH)], sems[j]
                )
            )
        for wr in writes:
            wr.wait()

    return gather(frames)


def kernel(frames):
    T = frames.shape[1]
    sel = _slow_indices(T)
    slow = _slow_gather_sc(frames, sel)
    fast = _fast_copy(frames)
    return (slow, fast)
